# baseline (speedup 1.0000x reference)
import jax
import jax.numpy as jnp
from jax import lax
from jax.experimental import pallas as pl
from jax.experimental.pallas import tpu as pltpu
import jax.experimental.pallas.tpu_sc as plsc

MAXLEN = 200
EMBED = 64
NUM_CORES = 2
NUM_SUBCORES = 16
NUM_WORKERS = NUM_CORES * NUM_SUBCORES
LANES = 16
NSLOT = 4
BSLAB = 128


def _body(xt_ref, tok_ref, pos_ref, out_ref, idx_v, rows_v, outt_v, pos_v,
          gsem, osem, isem):
  wid = lax.axis_index("s") * NUM_CORES + lax.axis_index("c")
  b0 = wid * BSLAB
  n_chunks = MAXLEN

  def idx_start(c, s):
    return pltpu.async_copy(
        xt_ref.at[c, pl.ds(b0, BSLAB)], idx_v.at[s], isem.at[s])

  def idx_wait(c, s):
    pltpu.make_async_copy(
        xt_ref.at[c, pl.ds(b0, BSLAB)], idx_v.at[s], isem.at[s]).wait()

  def gather_start(s):
    pltpu.async_copy(tok_ref.at[idx_v.at[s]], rows_v.at[s], gsem.at[s])

  def gather_wait(s):
    pltpu.make_async_copy(
        tok_ref.at[idx_v.at[s]], rows_v.at[s], gsem.at[s]).wait()

  def out_start(c, s):
    return pltpu.async_copy(outt_v.at[s], out_ref.at[c, :, wid], osem.at[s])

  def out_wait(c, s):
    pltpu.make_async_copy(
        outt_v.at[s], out_ref.at[c, :, wid], osem.at[s]).wait()

  pltpu.sync_copy(pos_ref, pos_v)

  row_idx = [lax.iota(jnp.int32, LANES) + g * LANES
             for g in range(BSLAB // LANES)]
  zf = jnp.zeros((LANES,), jnp.float32)
  zi = jnp.zeros((LANES,), jnp.int32)

  for s in range(NSLOT):
    idx_start(s, s)
  for s in range(2):
    idx_wait(s, s)
    gather_start(s)

  def outer(c4, carry):
    for s in range(NSLOT):
      c = c4 * NSLOT + s
      s2 = (s + 2) % NSLOT

      gather_wait(s)

      @pl.when(c >= NSLOT)
      def _():
        out_wait(c - NSLOT, s)

      rows_s = rows_v.at[s]
      outt_s = outt_v.at[s]
      cvec = c + zi

      def d4_body(d4, carry2):
        for dd in range(4):
          d = d4 * 4 + dd
          dvec = d + zi
          p = plsc.load_gather(pos_v, [cvec, dvec])
          dt = lax.div(d, 8)
          di = lax.rem(d, 8)
          for g in range(BSLAB // LANES):
            v = plsc.load_gather(rows_s, [row_idx[g], dvec])
            outt_s[dt, di, pl.ds(g * LANES, LANES)] = v + p
        return carry2

      lax.fori_loop(0, EMBED // 4, d4_body, None)

      out_start(c, s)

      @pl.when(c + 2 < n_chunks)
      def _():
        idx_wait(c + 2, s2)
        gather_start(s2)

      @pl.when(c + 4 < n_chunks)
      def _():
        idx_start(c + 4, s)
    return carry

  lax.fori_loop(0, n_chunks // NSLOT, outer, None)

  for s in range(NSLOT):
    out_wait(n_chunks - NSLOT + s, s)


def kernel(x, token_table, pos_table):
  batch, maxlen = x.shape
  xt = jnp.transpose(x.astype(jnp.int32))
  mesh = plsc.VectorSubcoreMesh(core_axis_name="c", subcore_axis_name="s")
  out5 = pl.kernel(
      _body,
      out_type=jax.ShapeDtypeStruct(
          (maxlen, EMBED // 8, batch // BSLAB, 8, BSLAB), jnp.float32),
      mesh=mesh,
      compiler_params=pltpu.CompilerParams(use_tc_tiling_on_sc=False, needs_layout_passes=False),
      scratch_types=[
          pltpu.VMEM((NSLOT, BSLAB), jnp.int32),
          pltpu.VMEM((NSLOT, BSLAB, EMBED), jnp.float32),
          pltpu.VMEM((NSLOT, EMBED // 8, 8, BSLAB), jnp.float32),
          pltpu.VMEM((MAXLEN, EMBED), jnp.float32),
          pltpu.SemaphoreType.DMA((NSLOT,)),
          pltpu.SemaphoreType.DMA((NSLOT,)),
          pltpu.SemaphoreType.DMA((NSLOT,)),
      ],
  )(xt, token_table, pos_table)
  return jnp.transpose(out5, (2, 4, 0, 1, 3)).reshape(batch, maxlen, EMBED)


# speedup vs baseline: 1.8942x; 1.8942x over previous
import jax
import jax.numpy as jnp
from jax import lax
from jax.experimental import pallas as pl
from jax.experimental.pallas import tpu as pltpu
import jax.experimental.pallas.tpu_sc as plsc

MAXLEN = 200
EMBED = 64
NUM_CORES = 2
NUM_SUBCORES = 16
NUM_WORKERS = NUM_CORES * NUM_SUBCORES
LANES = 16
NSLOT = 4
BSLAB = 128


def _body(xt_ref, tok_ref, pos_ref, out_ref, idx_v, rows_v, outt_v, pos_v,
          gsem, osem, isem):
  wid = lax.axis_index("s") * NUM_CORES + lax.axis_index("c")
  b0 = wid * BSLAB
  n_chunks = MAXLEN

  def idx_start(c, s):
    return pltpu.async_copy(
        xt_ref.at[c, pl.ds(b0, BSLAB)], idx_v.at[s], isem.at[s])

  def idx_wait(c, s):
    pltpu.make_async_copy(
        xt_ref.at[c, pl.ds(b0, BSLAB)], idx_v.at[s], isem.at[s]).wait()

  def gather_start(s):
    pltpu.async_copy(tok_ref.at[idx_v.at[s]], rows_v.at[s], gsem.at[s])

  def gather_wait(s):
    pltpu.make_async_copy(
        tok_ref.at[idx_v.at[s]], rows_v.at[s], gsem.at[s]).wait()

  def out_start(c, s):
    return pltpu.async_copy(outt_v.at[s], out_ref.at[c, :, wid], osem.at[s])

  def out_wait(c, s):
    pltpu.make_async_copy(
        outt_v.at[s], out_ref.at[c, :, wid], osem.at[s]).wait()

  pltpu.sync_copy(pos_ref, pos_v)

  row_idx = [lax.iota(jnp.int32, LANES) + g * LANES
             for g in range(BSLAB // LANES)]
  zf = jnp.zeros((LANES,), jnp.float32)
  zi = jnp.zeros((LANES,), jnp.int32)

  for s in range(NSLOT):
    idx_start(s, s)
  for s in range(2):
    idx_wait(s, s)
    gather_start(s)

  def outer(c4, carry):
    for s in range(NSLOT):
      c = c4 * NSLOT + s
      s2 = (s + 2) % NSLOT

      gather_wait(s)

      @pl.when(c >= NSLOT)
      def _():
        out_wait(c - NSLOT, s)

      rows_s = rows_v.at[s]
      outt_s = outt_v.at[s]
      cvec = c + zi

      @plsc.parallel_loop(0, EMBED, unroll=4)
      def _(d):
        dvec = d + zi
        p = plsc.load_gather(pos_v, [cvec, dvec])
        dt = lax.div(d, 8)
        di = lax.rem(d, 8)
        for g in range(BSLAB // LANES):
          v = plsc.load_gather(rows_s, [row_idx[g], dvec])
          outt_s[dt, di, pl.ds(g * LANES, LANES)] = v + p

      out_start(c, s)

      @pl.when(c + 2 < n_chunks)
      def _():
        idx_wait(c + 2, s2)
        gather_start(s2)

      @pl.when(c + 4 < n_chunks)
      def _():
        idx_start(c + 4, s)
    return carry

  lax.fori_loop(0, n_chunks // NSLOT, outer, None)

  for s in range(NSLOT):
    out_wait(n_chunks - NSLOT + s, s)


def kernel(x, token_table, pos_table):
  batch, maxlen = x.shape
  xt = jnp.transpose(x.astype(jnp.int32))
  mesh = plsc.VectorSubcoreMesh(core_axis_name="c", subcore_axis_name="s")
  out5 = pl.kernel(
      _body,
      out_type=jax.ShapeDtypeStruct(
          (maxlen, EMBED // 8, batch // BSLAB, 8, BSLAB), jnp.float32),
      mesh=mesh,
      compiler_params=pltpu.CompilerParams(use_tc_tiling_on_sc=False, needs_layout_passes=False),
      scratch_types=[
          pltpu.VMEM((NSLOT, BSLAB), jnp.int32),
          pltpu.VMEM((NSLOT, BSLAB, EMBED), jnp.float32),
          pltpu.VMEM((NSLOT, EMBED // 8, 8, BSLAB), jnp.float32),
          pltpu.VMEM((MAXLEN, EMBED), jnp.float32),
          pltpu.SemaphoreType.DMA((NSLOT,)),
          pltpu.SemaphoreType.DMA((NSLOT,)),
          pltpu.SemaphoreType.DMA((NSLOT,)),
      ],
  )(xt, token_table, pos_table)
  return jnp.transpose(out5, (2, 4, 0, 1, 3)).reshape(batch, maxlen, EMBED)


# speedup vs baseline: 4.9695x; 2.6235x over previous
import jax
import jax.numpy as jnp
from jax import lax
from jax.experimental import pallas as pl
from jax.experimental.pallas import tpu as pltpu
import jax.experimental.pallas.tpu_sc as plsc

MAXLEN = 200
EMBED = 64
NUM_CORES = 2
NUM_SUBCORES = 16
NUM_WORKERS = NUM_CORES * NUM_SUBCORES
LANES = 16
NSLOT = 4
BSLAB = 128


def _body(xt_ref, tok_ref, pos_ref, out_ref, idx_v, rows_v, outt_v, pos_v,
          gsem, osem, isem):
  wid = lax.axis_index("s") * NUM_CORES + lax.axis_index("c")
  b0 = wid * BSLAB
  n_chunks = MAXLEN

  def idx_start(c, s):
    return pltpu.async_copy(
        xt_ref.at[c, pl.ds(b0, BSLAB)], idx_v.at[s], isem.at[s])

  def idx_wait(c, s):
    pltpu.make_async_copy(
        xt_ref.at[c, pl.ds(b0, BSLAB)], idx_v.at[s], isem.at[s]).wait()

  def gather_start(s):
    pltpu.async_copy(tok_ref.at[idx_v.at[s]], rows_v.at[s], gsem.at[s])

  def gather_wait(s):
    pltpu.make_async_copy(
        tok_ref.at[idx_v.at[s]], rows_v.at[s], gsem.at[s]).wait()

  def out_start(c, s):
    return pltpu.async_copy(
        outt_v.at[s, :, :, pl.ds(0, BSLAB)], out_ref.at[c, :, wid],
        osem.at[s])

  def out_wait(c, s):
    pltpu.make_async_copy(
        outt_v.at[s, :, :, pl.ds(0, BSLAB)], out_ref.at[c, :, wid],
        osem.at[s]).wait()

  pltpu.sync_copy(pos_ref, pos_v)

  zi = jnp.zeros((LANES,), jnp.int32)
  dts, dis = [], []
  for g in range(EMBED // LANES):
    dg = lax.iota(jnp.int32, LANES) + g * LANES
    dts.append(dg // 8)
    dis.append(lax.rem(dg, 8))

  for s in range(NSLOT):
    idx_start(s, s)
  for s in range(2):
    idx_wait(s, s)
    gather_start(s)

  def outer(c4, carry):
    for s in range(NSLOT):
      c = c4 * NSLOT + s
      s2 = (s + 2) % NSLOT

      gather_wait(s)

      @pl.when(c >= NSLOT)
      def _():
        out_wait(c - NSLOT, s)

      rows_s = rows_v.at[s]
      outt_s = outt_v.at[s]
      pvecs = [pos_v[c, pl.ds(g * LANES, LANES)]
               for g in range(EMBED // LANES)]

      @plsc.parallel_loop(0, BSLAB, unroll=8)
      def _(j):
        col = j + zi
        for g in range(EMBED // LANES):
          v = rows_s[j, pl.ds(g * LANES, LANES)]
          plsc.store_scatter(outt_s, [dts[g], dis[g], col], v + pvecs[g])

      out_start(c, s)

      @pl.when(c + 2 < n_chunks)
      def _():
        idx_wait(c + 2, s2)
        gather_start(s2)

      @pl.when(c + 4 < n_chunks)
      def _():
        idx_start(c + 4, s)
    return carry

  lax.fori_loop(0, n_chunks // NSLOT, outer, None)

  for s in range(NSLOT):
    out_wait(n_chunks - NSLOT + s, s)


def kernel(x, token_table, pos_table):
  batch, maxlen = x.shape
  xt = jnp.transpose(x.astype(jnp.int32))
  mesh = plsc.VectorSubcoreMesh(core_axis_name="c", subcore_axis_name="s")
  out5 = pl.kernel(
      _body,
      out_type=jax.ShapeDtypeStruct(
          (maxlen, EMBED // 8, batch // BSLAB, 8, BSLAB), jnp.float32),
      mesh=mesh,
      compiler_params=pltpu.CompilerParams(use_tc_tiling_on_sc=False, needs_layout_passes=False),
      scratch_types=[
          pltpu.VMEM((NSLOT, BSLAB), jnp.int32),
          pltpu.VMEM((NSLOT, BSLAB, EMBED), jnp.float32),
          pltpu.VMEM((NSLOT, EMBED // 8, 8, BSLAB + 1), jnp.float32),
          pltpu.VMEM((MAXLEN, EMBED), jnp.float32),
          pltpu.SemaphoreType.DMA((NSLOT,)),
          pltpu.SemaphoreType.DMA((NSLOT,)),
          pltpu.SemaphoreType.DMA((NSLOT,)),
      ],
  )(xt, token_table, pos_table)
  return jnp.transpose(out5, (2, 4, 0, 1, 3)).reshape(batch, maxlen, EMBED)


# speedup vs baseline: 4.9732x; 1.0007x over previous
import jax
import jax.numpy as jnp
from jax import lax
from jax.experimental import pallas as pl
from jax.experimental.pallas import tpu as pltpu
import jax.experimental.pallas.tpu_sc as plsc

MAXLEN = 200
EMBED = 64
NUM_CORES = 2
NUM_SUBCORES = 16
NUM_WORKERS = NUM_CORES * NUM_SUBCORES
LANES = 16
NSLOT = 4
BSLAB = 128


def _body(xt_ref, tok_ref, pos_ref, out_ref, idx_v, rows_v, outt_v, pos_v,
          gsem, osem, isem):
  wid = lax.axis_index("s") * NUM_CORES + lax.axis_index("c")
  n_chunks = MAXLEN

  def idx_start(c, s):
    return pltpu.async_copy(
        xt_ref.at[c // 8, wid, lax.rem(c, 8)], idx_v.at[s], isem.at[s])

  def idx_wait(c, s):
    pltpu.make_async_copy(
        xt_ref.at[c // 8, wid, lax.rem(c, 8)], idx_v.at[s], isem.at[s]).wait()

  def gather_start(s):
    pltpu.async_copy(tok_ref.at[idx_v.at[s]], rows_v.at[s], gsem.at[s])

  def gather_wait(s):
    pltpu.make_async_copy(
        tok_ref.at[idx_v.at[s]], rows_v.at[s], gsem.at[s]).wait()

  def out_start(c, s):
    return pltpu.async_copy(
        outt_v.at[s, :, :, pl.ds(0, BSLAB)], out_ref.at[c, :, wid],
        osem.at[s])

  def out_wait(c, s):
    pltpu.make_async_copy(
        outt_v.at[s, :, :, pl.ds(0, BSLAB)], out_ref.at[c, :, wid],
        osem.at[s]).wait()

  pltpu.sync_copy(pos_ref, pos_v)

  zi = jnp.zeros((LANES,), jnp.int32)
  dts, dis = [], []
  for g in range(EMBED // LANES):
    dg = lax.iota(jnp.int32, LANES) + g * LANES
    dts.append(dg // 8)
    dis.append(lax.rem(dg, 8))

  for s in range(NSLOT):
    idx_start(s, s)
  for s in range(2):
    idx_wait(s, s)
    gather_start(s)

  def outer(c4, carry):
    for s in range(NSLOT):
      c = c4 * NSLOT + s
      s2 = (s + 2) % NSLOT

      gather_wait(s)

      @pl.when(c >= NSLOT)
      def _():
        out_wait(c - NSLOT, s)

      rows_s = rows_v.at[s]
      outt_s = outt_v.at[s]
      pvecs = [pos_v[c, pl.ds(g * LANES, LANES)]
               for g in range(EMBED // LANES)]

      @plsc.parallel_loop(0, BSLAB, unroll=8)
      def _(j):
        col = j + zi
        for g in range(EMBED // LANES):
          v = rows_s[j, pl.ds(g * LANES, LANES)]
          plsc.store_scatter(outt_s, [dts[g], dis[g], col], v + pvecs[g])

      out_start(c, s)

      @pl.when(c + 2 < n_chunks)
      def _():
        idx_wait(c + 2, s2)
        gather_start(s2)

      @pl.when(c + 4 < n_chunks)
      def _():
        idx_start(c + 4, s)
    return carry

  lax.fori_loop(0, n_chunks // NSLOT, outer, None)

  for s in range(NSLOT):
    out_wait(n_chunks - NSLOT + s, s)


def kernel(x, token_table, pos_table):
  batch, maxlen = x.shape
  xt4 = (jnp.transpose(x.astype(jnp.int32))
         .reshape(maxlen // 8, 8, batch // BSLAB, BSLAB)
         .transpose(0, 2, 1, 3))
  mesh = plsc.VectorSubcoreMesh(core_axis_name="c", subcore_axis_name="s")
  out5 = pl.kernel(
      _body,
      out_type=jax.ShapeDtypeStruct(
          (maxlen, EMBED // 8, batch // BSLAB, 8, BSLAB), jnp.float32),
      mesh=mesh,
      compiler_params=pltpu.CompilerParams(use_tc_tiling_on_sc=False, needs_layout_passes=False),
      scratch_types=[
          pltpu.VMEM((NSLOT, BSLAB), jnp.int32),
          pltpu.VMEM((NSLOT, BSLAB, EMBED), jnp.float32),
          pltpu.VMEM((NSLOT, EMBED // 8, 8, BSLAB + 1), jnp.float32),
          pltpu.VMEM((MAXLEN, EMBED), jnp.float32),
          pltpu.SemaphoreType.DMA((NSLOT,)),
          pltpu.SemaphoreType.DMA((NSLOT,)),
          pltpu.SemaphoreType.DMA((NSLOT,)),
      ],
  )(xt4, token_table, pos_table)
  return jnp.transpose(out5, (2, 4, 0, 1, 3)).reshape(batch, maxlen, EMBED)


# speedup vs baseline: 5.0322x; 1.0119x over previous
import jax
import jax.numpy as jnp
from jax import lax
from jax.experimental import pallas as pl
from jax.experimental.pallas import tpu as pltpu
import jax.experimental.pallas.tpu_sc as plsc

MAXLEN = 200
EMBED = 64
NUM_CORES = 2
NUM_SUBCORES = 16
NUM_WORKERS = NUM_CORES * NUM_SUBCORES
LANES = 16
NSLOT = 5
BSLAB = 128


def _body(xt_ref, tok_ref, pos_ref, out_ref, idx_v, rows_v, outt_v, pos_v,
          gsem, osem, isem):
  wid = lax.axis_index("s") * NUM_CORES + lax.axis_index("c")
  n_chunks = MAXLEN

  def idx_start(c, s):
    return pltpu.async_copy(
        xt_ref.at[c // 8, wid, lax.rem(c, 8)], idx_v.at[s], isem.at[s])

  def idx_wait(c, s):
    pltpu.make_async_copy(
        xt_ref.at[c // 8, wid, lax.rem(c, 8)], idx_v.at[s], isem.at[s]).wait()

  def gather_start(s):
    pltpu.async_copy(tok_ref.at[idx_v.at[s]], rows_v.at[s], gsem.at[s])

  def gather_wait(s):
    pltpu.make_async_copy(
        tok_ref.at[idx_v.at[s]], rows_v.at[s], gsem.at[s]).wait()

  def out_start(c, s):
    return pltpu.async_copy(
        outt_v.at[s, :, :, pl.ds(0, BSLAB)], out_ref.at[c, :, wid],
        osem.at[s])

  def out_wait(c, s):
    pltpu.make_async_copy(
        outt_v.at[s, :, :, pl.ds(0, BSLAB)], out_ref.at[c, :, wid],
        osem.at[s]).wait()

  pltpu.sync_copy(pos_ref, pos_v)

  zi = jnp.zeros((LANES,), jnp.int32)
  dts, dis = [], []
  for g in range(EMBED // LANES):
    dg = lax.iota(jnp.int32, LANES) + g * LANES
    dts.append(dg // 8)
    dis.append(lax.rem(dg, 8))

  for s in range(NSLOT):
    idx_start(s, s)
  for s in range(2):
    idx_wait(s, s)
    gather_start(s)

  def outer(c4, carry):
    for s in range(NSLOT):
      c = c4 * NSLOT + s
      s2 = (s + 2) % NSLOT

      gather_wait(s)

      @pl.when(c >= NSLOT)
      def _():
        out_wait(c - NSLOT, s)

      rows_s = rows_v.at[s]
      outt_s = outt_v.at[s]
      pvecs = [pos_v[c, pl.ds(g * LANES, LANES)]
               for g in range(EMBED // LANES)]

      @plsc.parallel_loop(0, BSLAB, unroll=16)
      def _(j):
        col = j + zi
        for g in range(EMBED // LANES):
          v = rows_s[j, pl.ds(g * LANES, LANES)]
          plsc.store_scatter(outt_s, [dts[g], dis[g], col], v + pvecs[g])

      out_start(c, s)

      @pl.when(c + 2 < n_chunks)
      def _():
        idx_wait(c + 2, s2)
        gather_start(s2)

      @pl.when(c + NSLOT < n_chunks)
      def _():
        idx_start(c + NSLOT, s)
    return carry

  lax.fori_loop(0, n_chunks // NSLOT, outer, None)

  for s in range(NSLOT):
    out_wait(n_chunks - NSLOT + s, s)


def kernel(x, token_table, pos_table):
  batch, maxlen = x.shape
  xt4 = (jnp.transpose(x.astype(jnp.int32))
         .reshape(maxlen // 8, 8, batch // BSLAB, BSLAB)
         .transpose(0, 2, 1, 3))
  mesh = plsc.VectorSubcoreMesh(core_axis_name="c", subcore_axis_name="s")
  out5 = pl.kernel(
      _body,
      out_type=jax.ShapeDtypeStruct(
          (maxlen, EMBED // 8, batch // BSLAB, 8, BSLAB), jnp.float32),
      mesh=mesh,
      compiler_params=pltpu.CompilerParams(use_tc_tiling_on_sc=False, needs_layout_passes=False),
      scratch_types=[
          pltpu.VMEM((NSLOT, BSLAB), jnp.int32),
          pltpu.VMEM((NSLOT, BSLAB, EMBED), jnp.float32),
          pltpu.VMEM((NSLOT, EMBED // 8, 8, BSLAB + 1), jnp.float32),
          pltpu.VMEM((MAXLEN, EMBED), jnp.float32),
          pltpu.SemaphoreType.DMA((NSLOT,)),
          pltpu.SemaphoreType.DMA((NSLOT,)),
          pltpu.SemaphoreType.DMA((NSLOT,)),
      ],
  )(xt4, token_table, pos_table)
  return jnp.transpose(out5, (2, 4, 0, 1, 3)).reshape(batch, maxlen, EMBED)


# speedup vs baseline: 5.8067x; 1.1539x over previous
import jax
import jax.numpy as jnp
from jax import lax
from jax.experimental import pallas as pl
from jax.experimental.pallas import tpu as pltpu
import jax.experimental.pallas.tpu_sc as plsc

MAXLEN = 200
EMBED = 64
NUM_CORES = 2
NUM_SUBCORES = 16
NUM_WORKERS = NUM_CORES * NUM_SUBCORES
LANES = 16
NSLOT = 5
BSLAB = 128


def _body(xt_ref, tok_ref, pos_ref, out_ref, idx_v, rows_v, outt_v, pos_v,
          gsem, osem, isem):
  wid = lax.axis_index("s") * NUM_CORES + lax.axis_index("c")
  n_chunks = MAXLEN

  def idx_start(c, s):
    return pltpu.async_copy(
        xt_ref.at[c // 8, wid, lax.rem(c, 8)], idx_v.at[s], isem.at[s])

  def idx_wait(c, s):
    pltpu.make_async_copy(
        xt_ref.at[c // 8, wid, lax.rem(c, 8)], idx_v.at[s], isem.at[s]).wait()

  def gather_start(s):
    pltpu.async_copy(tok_ref.at[idx_v.at[s]], rows_v.at[s], gsem.at[s])

  def gather_wait(s):
    pltpu.make_async_copy(
        tok_ref.at[idx_v.at[s]], rows_v.at[s], gsem.at[s]).wait()

  def out_start(c, s):
    return pltpu.async_copy(
        outt_v.at[s, :, :, pl.ds(0, BSLAB)], out_ref.at[c, :, wid],
        osem.at[s])

  def out_wait(c, s):
    pltpu.make_async_copy(
        outt_v.at[s, :, :, pl.ds(0, BSLAB)], out_ref.at[c, :, wid],
        osem.at[s]).wait()

  pltpu.sync_copy(pos_ref, pos_v)

  zi = jnp.zeros((LANES,), jnp.int32)
  dts, dis = [], []
  for g in range(EMBED // LANES):
    dg = lax.iota(jnp.int32, LANES) + g * LANES
    dts.append(dg // 8)
    dis.append(lax.rem(dg, 8))

  for s in range(NSLOT):
    idx_start(s, s)
  for s in range(2):
    idx_wait(s, s)
    gather_start(s)

  def outer(c4, carry):
    for s in range(NSLOT):
      c = c4 * NSLOT + s
      s2 = (s + 2) % NSLOT

      @pl.when(c + 2 < n_chunks)
      def _():
        idx_wait(c + 2, s2)
        gather_start(s2)

      gather_wait(s)

      @pl.when(c >= NSLOT)
      def _():
        out_wait(c - NSLOT, s)

      rows_s = rows_v.at[s]
      outt_s = outt_v.at[s]
      pvecs = [pos_v[c, pl.ds(g * LANES, LANES)]
               for g in range(EMBED // LANES)]

      @plsc.parallel_loop(0, BSLAB, unroll=16)
      def _(j):
        col = j + zi
        for g in range(EMBED // LANES):
          v = rows_s[j, pl.ds(g * LANES, LANES)]
          plsc.store_scatter(outt_s, [dts[g], dis[g], col], v + pvecs[g])

      out_start(c, s)

      @pl.when(c + NSLOT < n_chunks)
      def _():
        idx_start(c + NSLOT, s)
    return carry

  lax.fori_loop(0, n_chunks // NSLOT, outer, None)

  for s in range(NSLOT):
    out_wait(n_chunks - NSLOT + s, s)


def kernel(x, token_table, pos_table):
  batch, maxlen = x.shape
  xt4 = (jnp.transpose(x.astype(jnp.int32))
         .reshape(maxlen // 8, 8, batch // BSLAB, BSLAB)
         .transpose(0, 2, 1, 3))
  mesh = plsc.VectorSubcoreMesh(core_axis_name="c", subcore_axis_name="s")
  out5 = pl.kernel(
      _body,
      out_type=jax.ShapeDtypeStruct(
          (maxlen, EMBED // 8, batch // BSLAB, 8, BSLAB), jnp.float32),
      mesh=mesh,
      compiler_params=pltpu.CompilerParams(use_tc_tiling_on_sc=False, needs_layout_passes=False),
      scratch_types=[
          pltpu.VMEM((NSLOT, BSLAB), jnp.int32),
          pltpu.VMEM((NSLOT, BSLAB, EMBED), jnp.float32),
          pltpu.VMEM((NSLOT, EMBED // 8, 8, BSLAB + 1), jnp.float32),
          pltpu.VMEM((MAXLEN, EMBED), jnp.float32),
          pltpu.SemaphoreType.DMA((NSLOT,)),
          pltpu.SemaphoreType.DMA((NSLOT,)),
          pltpu.SemaphoreType.DMA((NSLOT,)),
      ],
  )(xt4, token_table, pos_table)
  return jnp.transpose(out5, (2, 4, 0, 1, 3)).reshape(batch, maxlen, EMBED)
